# baseline (speedup 1.0000x reference)
import functools

import jax
import jax.numpy as jnp
from jax import lax
from jax.experimental import pallas as pl
from jax.experimental.pallas import tpu as pltpu
from jax.experimental.pallas import tpu_sc as plsc

N = 10000
E = 320000
D_IN = 128
D_H = 128
D_Z = 32
EPS = 1e-5

NC = 2
NS = 16
NW = NC * NS
EP = 327680
EPW = EP // NW
CH = 80
NCHUNK = EPW // CH
SLAB = 64
NSLAB = NCHUNK // SLAB

N2R = 10240
RPT = N2R // NS
DPT = N2R // NS

_MESH = dict(core_axis_name="c", subcore_axis_name="s", num_cores=NC,
             num_subcores=NS)


def _zero_fill(ref, rows, cols):
    zv = jnp.zeros((16,), jnp.float32)
    per_row = cols // 16

    def body(i, _):
        r = i // per_row
        c = (i % per_row) * 16
        ref[r, pl.ds(c, 16)] = zv
        return 0

    lax.fori_loop(0, rows * per_row, body, 0)


def _make_agg(d):
    mesh = plsc.VectorSubcoreMesh(**_MESH)

    @functools.partial(
        pl.kernel,
        out_type=jax.ShapeDtypeStruct((NC * N2R, d), jnp.float32),
        mesh=mesh,
        scratch_types=[
            pltpu.VMEM((SLAB, CH), jnp.int32),
            pltpu.VMEM((SLAB, CH), jnp.int32),
            pltpu.VMEM((CH, d), jnp.float32),
            pltpu.VMEM((CH, d), jnp.float32),
            pltpu.VMEM_SHARED((N2R, d), jnp.float32),
            pltpu.SemaphoreType.DMA,
            pltpu.SemaphoreType.DMA,
        ],
    )
    def agg(hs_hbm, src_hbm, dst_hbm, out_hbm,
            src_v, dst_v, bufa, bufb, acc, sema, semb):
        cid = lax.axis_index("c")
        sid = lax.axis_index("s")
        wid = sid * NC + cid

        _zero_fill(bufa, CH, d)
        for j in range(RPT // CH):
            pltpu.sync_copy(bufa, acc.at[pl.ds(sid * RPT + j * CH, CH)])
        plsc.subcore_barrier()

        bufs = (bufa, bufb)
        sems = (sema, semb)

        for slab in range(NSLAB):
            pltpu.sync_copy(src_hbm.at[wid, slab], src_v)
            pltpu.sync_copy(dst_hbm.at[wid, slab], dst_v)

            pltpu.async_copy(hs_hbm.at[src_v.at[0]], bufa, sema)
            pltpu.async_copy(hs_hbm.at[src_v.at[1]], bufb, semb)

            def pair(p, _):
                g = 2 * p
                for b in range(2):
                    i = g + b
                    pltpu.make_async_copy(hs_hbm.at[src_v.at[i]], bufs[b],
                                          sems[b]).wait()
                    pltpu.sync_copy(bufs[b], acc.at[dst_v.at[i]], add=True)
                    pltpu.async_copy(hs_hbm.at[src_v.at[i + 2]], bufs[b],
                                     sems[b])
                return 0

            lax.fori_loop(0, (SLAB - 2) // 2, pair, 0)
            for b in range(2):
                i = SLAB - 2 + b
                pltpu.make_async_copy(hs_hbm.at[src_v.at[i]], bufs[b],
                                      sems[b]).wait()
                pltpu.sync_copy(bufs[b], acc.at[dst_v.at[i]], add=True)

        plsc.subcore_barrier()
        pltpu.sync_copy(acc.at[pl.ds(sid * RPT, RPT)],
                        out_hbm.at[pl.ds(cid * N2R + sid * RPT, RPT)])

    return agg


_agg_h = _make_agg(D_H)


def _make_deg():
    mesh = plsc.VectorSubcoreMesh(**_MESH)

    @functools.partial(
        pl.kernel,
        out_type=jax.ShapeDtypeStruct((NC * N2R,), jnp.float32),
        mesh=mesh,
        scratch_types=[
            pltpu.VMEM((NCHUNK, CH), jnp.int32),
            pltpu.VMEM((CH,), jnp.float32),
            pltpu.VMEM((DPT,), jnp.float32),
            pltpu.VMEM_SHARED((N2R,), jnp.float32),
        ],
    )
    def deg_k(dst_hbm, out_hbm, dst_v, ones_v, zbuf, acc):
        cid = lax.axis_index("c")
        sid = lax.axis_index("s")
        wid = sid * NC + cid

        ov = jnp.ones((16,), jnp.float32)
        for k in range(CH // 16):
            ones_v[pl.ds(k * 16, 16)] = ov
        zv = jnp.zeros((16,), jnp.float32)

        def zb(i, _):
            zbuf[pl.ds(i * 16, 16)] = zv
            return 0

        lax.fori_loop(0, DPT // 16, zb, 0)
        pltpu.sync_copy(zbuf, acc.at[pl.ds(sid * DPT, DPT)])
        pltpu.sync_copy(dst_hbm.at[wid], dst_v)
        plsc.subcore_barrier()

        def body(i, _):
            pltpu.sync_copy(ones_v, acc.at[dst_v.at[i]], add=True)
            return 0

        lax.fori_loop(0, NCHUNK, body, 0)

        plsc.subcore_barrier()
        pltpu.sync_copy(acc.at[pl.ds(sid * DPT, DPT)],
                        out_hbm.at[pl.ds(cid * N2R + sid * DPT, DPT)])

    return deg_k


_deg_k = _make_deg()

R_BLK = 2000
GRID = N // R_BLK


def _tc1_body(x_ref, g_ref, b_ref, w_ref, d0_ref, d1_ref, hs_ref, dinv_ref):
    x = x_ref[...]
    mu = jnp.mean(x, axis=1, keepdims=True)
    xc = x - mu
    var = jnp.mean(xc * xc, axis=1, keepdims=True)
    h = xc * lax.rsqrt(var + EPS) * g_ref[...] + b_ref[...]
    dinv = lax.rsqrt(d0_ref[...] + d1_ref[...] + 1.0)
    hs = jnp.dot(h, w_ref[...], preferred_element_type=jnp.float32) * dinv
    hs_ref[...] = hs
    dinv_ref[...] = dinv


def _tc_mid_body(s0_ref, s1_ref, hs_ref, dinv_ref, b_ref, w_ref, out_ref):
    dinv = dinv_ref[...]
    h = (s0_ref[...] + s1_ref[...] + hs_ref[...]) * dinv + b_ref[...]
    h = jnp.maximum(h, 0.0)
    out_ref[...] = jnp.dot(h, w_ref[...],
                           preferred_element_type=jnp.float32) * dinv


def _tc_g3_body(s0_ref, s1_ref, hs_ref, dinv_ref, b_ref, g3_ref):
    dinv = dinv_ref[...]
    h = (s0_ref[...] + s1_ref[...] + hs_ref[...]) * dinv + b_ref[...]
    g3_ref[...] = jnp.maximum(h, 0.0) * dinv


def _tc_fin_body(t0_ref, t1_ref, g3_ref, dinv_ref, b_ref, w_ref, z_ref):
    s = t0_ref[...] + t1_ref[...] + g3_ref[...]
    z_ref[...] = (jnp.dot(s, w_ref[...], preferred_element_type=jnp.float32)
                  * dinv_ref[...] + b_ref[...])


def _row_spec(cols):
    return pl.BlockSpec((R_BLK, cols), lambda i: (i, 0))


def _full_spec(rows, cols):
    return pl.BlockSpec((rows, cols), lambda i: (0, 0))


def _tc1(x, g, b, w, d0, d1):
    return pl.pallas_call(
        _tc1_body,
        grid=(GRID,),
        in_specs=[_row_spec(D_IN), _full_spec(1, D_IN), _full_spec(1, D_IN),
                  _full_spec(D_IN, D_H), _row_spec(1), _row_spec(1)],
        out_specs=[_row_spec(D_H), _row_spec(1)],
        out_shape=[jax.ShapeDtypeStruct((N, D_H), jnp.float32),
                   jax.ShapeDtypeStruct((N, 1), jnp.float32)],
    )(x, g, b, w, d0, d1)


def _tc_mid(s0, s1, hs, dinv, b, w):
    dout = w.shape[1]
    return pl.pallas_call(
        _tc_mid_body,
        grid=(GRID,),
        in_specs=[_row_spec(D_H), _row_spec(D_H), _row_spec(D_H), _row_spec(1),
                  _full_spec(1, D_H), _full_spec(D_H, dout)],
        out_specs=_row_spec(dout),
        out_shape=jax.ShapeDtypeStruct((N, dout), jnp.float32),
    )(s0, s1, hs, dinv, b, w)


def _tc_g3(s0, s1, hs, dinv, b):
    return pl.pallas_call(
        _tc_g3_body,
        grid=(GRID,),
        in_specs=[_row_spec(D_H), _row_spec(D_H), _row_spec(D_H), _row_spec(1),
                  _full_spec(1, D_H)],
        out_specs=_row_spec(D_H),
        out_shape=jax.ShapeDtypeStruct((N, D_H), jnp.float32),
    )(s0, s1, hs, dinv, b)


def _tc_fin(t0, t1, g3, dinv, b, w):
    return pl.pallas_call(
        _tc_fin_body,
        grid=(GRID,),
        in_specs=[_row_spec(D_H), _row_spec(D_H), _row_spec(D_H), _row_spec(1),
                  _full_spec(1, D_Z), _full_spec(D_H, D_Z)],
        out_specs=_row_spec(D_Z),
        out_shape=jax.ShapeDtypeStruct((N, D_Z), jnp.float32),
    )(t0, t1, g3, dinv, b, w)


def kernel(x, edge_index, ln_g, ln_b, W1, b1, W2, b2, W3, b3):
    npad = EP - E
    src_p = jnp.concatenate([edge_index[0], jnp.zeros((npad,), jnp.int32)])
    pad_dst = N + (jnp.arange(npad, dtype=jnp.int32) % (N2R - N))
    dst_p = jnp.concatenate([edge_index[1], pad_dst])
    src4 = src_p.reshape(NW, NSLAB, SLAB, CH)
    dst4 = dst_p.reshape(NW, NSLAB, SLAB, CH)
    dst3 = dst_p.reshape(NW, NCHUNK, CH)

    deg = _deg_k(dst3)
    d0 = deg[:N, None]
    d1 = deg[N2R:N2R + N, None]

    hs1, dinv = _tc1(x, ln_g.reshape(1, -1), ln_b.reshape(1, -1), W1, d0, d1)
    s1 = _agg_h(hs1, src4, dst4)
    hs2 = _tc_mid(s1[:N], s1[N2R:N2R + N], hs1, dinv, b1.reshape(1, -1), W2)
    s2 = _agg_h(hs2, src4, dst4)
    g3 = _tc_g3(s2[:N], s2[N2R:N2R + N], hs2, dinv, b2.reshape(1, -1))
    t3 = _agg_h(g3, src4, dst4)
    return _tc_fin(t3[:N], t3[N2R:N2R + N], g3, dinv, b3.reshape(1, -1), W3)


# speedup vs baseline: 2.3327x; 2.3327x over previous
import functools

import jax
import jax.numpy as jnp
from jax import lax
from jax.experimental import pallas as pl
from jax.experimental.pallas import tpu as pltpu
from jax.experimental.pallas import tpu_sc as plsc

N = 10000
E = 320000
D_IN = 128
D_H = 128
D_Z = 32
EPS = 1e-5

NC = 1
NS = 16
NW = NC * NS
EPW = E // NW
CH = 80
NCHUNK = EPW // CH
SLAB = 50
NSLAB = NCHUNK // SLAB

N2R = 10240
RPT = N2R // NS
DPT = N2R // NS

_MESH = dict(core_axis_name="c", subcore_axis_name="s", num_cores=NC,
             num_subcores=NS)


def _zero_fill(ref, rows, cols):
    zv = jnp.zeros((16,), jnp.float32)
    per_row = cols // 16

    def body(i, _):
        r = i // per_row
        c = (i % per_row) * 16
        ref[r, pl.ds(c, 16)] = zv
        return 0

    lax.fori_loop(0, rows * per_row, body, 0)


def _make_agg(d):
    mesh = plsc.VectorSubcoreMesh(**_MESH)

    @functools.partial(
        pl.kernel,
        out_type=jax.ShapeDtypeStruct((N2R, d), jnp.float32),
        mesh=mesh,
        scratch_types=[
            pltpu.VMEM((SLAB, CH), jnp.int32),
            pltpu.VMEM((SLAB, CH), jnp.int32),
            pltpu.VMEM((CH, d), jnp.float32),
            pltpu.VMEM((CH, d), jnp.float32),
            pltpu.VMEM((CH, d), jnp.float32),
            pltpu.VMEM_SHARED((N2R, d), jnp.float32),
            pltpu.SemaphoreType.DMA,
            pltpu.SemaphoreType.DMA,
            pltpu.SemaphoreType.DMA,
        ],
    )
    def agg(hs_hbm, src_hbm, dst_hbm, out_hbm,
            src_v, dst_v, bufa, bufb, bufc, acc, sema, semb, semc):
        sid = lax.axis_index("s")
        wid = sid

        _zero_fill(bufa, CH, d)
        for j in range(RPT // CH):
            pltpu.sync_copy(bufa, acc.at[pl.ds(sid * RPT + j * CH, CH)])
        plsc.subcore_barrier()

        bufs = (bufa, bufb, bufc)
        sems = (sema, semb, semc)

        def step(i, b, prefetch):
            pltpu.make_async_copy(hs_hbm.at[src_v.at[i]], bufs[b],
                                  sems[b]).wait()
            pltpu.sync_copy(bufs[b], acc.at[dst_v.at[i]], add=True)
            if prefetch:
                pltpu.async_copy(hs_hbm.at[src_v.at[i + 3]], bufs[b], sems[b])

        for slab in range(NSLAB):
            pltpu.sync_copy(src_hbm.at[wid, slab], src_v)
            pltpu.sync_copy(dst_hbm.at[wid, slab], dst_v)

            for b in range(3):
                pltpu.async_copy(hs_hbm.at[src_v.at[b]], bufs[b], sems[b])
            step(0, 0, True)
            step(1, 1, True)

            def triple(p, _):
                g = 2 + 3 * p
                for b in range(3):
                    step(g + b, (2 + b) % 3, True)
                return 0

            lax.fori_loop(0, (SLAB - 5) // 3, triple, 0)
            for i in range(SLAB - 3, SLAB):
                step(i, i % 3, False)

        plsc.subcore_barrier()
        pltpu.sync_copy(acc.at[pl.ds(sid * RPT, RPT)],
                        out_hbm.at[pl.ds(sid * RPT, RPT)])

    return agg


_agg_h = _make_agg(D_H)


def _make_deg():
    mesh = plsc.VectorSubcoreMesh(**_MESH)

    @functools.partial(
        pl.kernel,
        out_type=jax.ShapeDtypeStruct((N2R,), jnp.float32),
        mesh=mesh,
        scratch_types=[
            pltpu.VMEM((NCHUNK, CH), jnp.int32),
            pltpu.VMEM((CH,), jnp.float32),
            pltpu.VMEM((DPT,), jnp.float32),
            pltpu.VMEM_SHARED((N2R,), jnp.float32),
        ],
    )
    def deg_k(dst_hbm, out_hbm, dst_v, ones_v, zbuf, acc):
        sid = lax.axis_index("s")
        wid = sid

        ov = jnp.ones((16,), jnp.float32)
        for k in range(CH // 16):
            ones_v[pl.ds(k * 16, 16)] = ov
        zv = jnp.zeros((16,), jnp.float32)

        def zb(i, _):
            zbuf[pl.ds(i * 16, 16)] = zv
            return 0

        lax.fori_loop(0, DPT // 16, zb, 0)
        pltpu.sync_copy(zbuf, acc.at[pl.ds(sid * DPT, DPT)])
        pltpu.sync_copy(dst_hbm.at[wid], dst_v)
        plsc.subcore_barrier()

        def body(i, _):
            pltpu.sync_copy(ones_v, acc.at[dst_v.at[i]], add=True)
            return 0

        lax.fori_loop(0, NCHUNK, body, 0)

        plsc.subcore_barrier()
        pltpu.sync_copy(acc.at[pl.ds(sid * DPT, DPT)],
                        out_hbm.at[pl.ds(sid * DPT, DPT)])

    return deg_k


_deg_k = _make_deg()

R_BLK = 2000
GRID = N // R_BLK


def _tc1_body(x_ref, g_ref, b_ref, w_ref, d0_ref, hs_ref, dinv_ref):
    x = x_ref[...]
    mu = jnp.mean(x, axis=1, keepdims=True)
    xc = x - mu
    var = jnp.mean(xc * xc, axis=1, keepdims=True)
    h = xc * lax.rsqrt(var + EPS) * g_ref[...] + b_ref[...]
    dinv = lax.rsqrt(d0_ref[...] + 1.0)
    hs = jnp.dot(h, w_ref[...], preferred_element_type=jnp.float32) * dinv
    hs_ref[...] = hs
    dinv_ref[...] = dinv


def _tc_mid_body(s0_ref, hs_ref, dinv_ref, b_ref, w_ref, out_ref):
    dinv = dinv_ref[...]
    h = (s0_ref[...] + hs_ref[...]) * dinv + b_ref[...]
    h = jnp.maximum(h, 0.0)
    out_ref[...] = jnp.dot(h, w_ref[...],
                           preferred_element_type=jnp.float32) * dinv


def _tc_g3_body(s0_ref, hs_ref, dinv_ref, b_ref, g3_ref):
    dinv = dinv_ref[...]
    h = (s0_ref[...] + hs_ref[...]) * dinv + b_ref[...]
    g3_ref[...] = jnp.maximum(h, 0.0) * dinv


def _tc_fin_body(t_ref, g3_ref, dinv_ref, b_ref, w_ref, z_ref):
    s = t_ref[...] + g3_ref[...]
    z_ref[...] = (jnp.dot(s, w_ref[...], preferred_element_type=jnp.float32)
                  * dinv_ref[...] + b_ref[...])


def _row_spec(cols):
    return pl.BlockSpec((R_BLK, cols), lambda i: (i, 0))


def _full_spec(rows, cols):
    return pl.BlockSpec((rows, cols), lambda i: (0, 0))


def _tc1(x, g, b, w, d0):
    return pl.pallas_call(
        _tc1_body,
        grid=(GRID,),
        in_specs=[_row_spec(D_IN), _full_spec(1, D_IN), _full_spec(1, D_IN),
                  _full_spec(D_IN, D_H), _row_spec(1)],
        out_specs=[_row_spec(D_H), _row_spec(1)],
        out_shape=[jax.ShapeDtypeStruct((N, D_H), jnp.float32),
                   jax.ShapeDtypeStruct((N, 1), jnp.float32)],
    )(x, g, b, w, d0)


def _tc_mid(s0, hs, dinv, b, w):
    dout = w.shape[1]
    return pl.pallas_call(
        _tc_mid_body,
        grid=(GRID,),
        in_specs=[_row_spec(D_H), _row_spec(D_H), _row_spec(1),
                  _full_spec(1, D_H), _full_spec(D_H, dout)],
        out_specs=_row_spec(dout),
        out_shape=jax.ShapeDtypeStruct((N, dout), jnp.float32),
    )(s0, hs, dinv, b, w)


def _tc_g3(s0, hs, dinv, b):
    return pl.pallas_call(
        _tc_g3_body,
        grid=(GRID,),
        in_specs=[_row_spec(D_H), _row_spec(D_H), _row_spec(1),
                  _full_spec(1, D_H)],
        out_specs=_row_spec(D_H),
        out_shape=jax.ShapeDtypeStruct((N, D_H), jnp.float32),
    )(s0, hs, dinv, b)


def _tc_fin(t, g3, dinv, b, w):
    return pl.pallas_call(
        _tc_fin_body,
        grid=(GRID,),
        in_specs=[_row_spec(D_H), _row_spec(D_H), _row_spec(1),
                  _full_spec(1, D_Z), _full_spec(D_H, D_Z)],
        out_specs=_row_spec(D_Z),
        out_shape=jax.ShapeDtypeStruct((N, D_Z), jnp.float32),
    )(t, g3, dinv, b, w)


def kernel(x, edge_index, ln_g, ln_b, W1, b1, W2, b2, W3, b3):
    src4 = edge_index[0].reshape(NW, NSLAB, SLAB, CH)
    dst4 = edge_index[1].reshape(NW, NSLAB, SLAB, CH)
    dst3 = edge_index[1].reshape(NW, NCHUNK, CH)

    deg = _deg_k(dst3)
    d0 = deg[:N, None]

    hs1, dinv = _tc1(x, ln_g.reshape(1, -1), ln_b.reshape(1, -1), W1, d0)
    s1 = _agg_h(hs1, src4, dst4)
    hs2 = _tc_mid(s1[:N], hs1, dinv, b1.reshape(1, -1), W2)
    s2 = _agg_h(hs2, src4, dst4)
    g3 = _tc_g3(s2[:N], hs2, dinv, b2.reshape(1, -1))
    t3 = _agg_h(g3, src4, dst4)
    return _tc_fin(t3[:N], g3, dinv, b3.reshape(1, -1), W3)


# speedup vs baseline: 2.3354x; 1.0012x over previous
import functools

import jax
import jax.numpy as jnp
from jax import lax
from jax.experimental import pallas as pl
from jax.experimental.pallas import tpu as pltpu
from jax.experimental.pallas import tpu_sc as plsc

N = 10000
E = 320000
D_IN = 128
D_H = 128
D_Z = 32
EPS = 1e-5

NC = 1
NS = 16
NW = NC * NS
EPW = E // NW
CH = 80
NCHUNK = EPW // CH
SLAB = 50
NSLAB = NCHUNK // SLAB

N2R = 10240
RPT = N2R // NS
DPT = N2R // NS

_MESH = dict(core_axis_name="c", subcore_axis_name="s", num_cores=NC,
             num_subcores=NS)


def _zero_fill(ref, rows, cols):
    zv = jnp.zeros((16,), jnp.float32)
    per_row = cols // 16

    def body(i, _):
        r = i // per_row
        c = (i % per_row) * 16
        ref[r, pl.ds(c, 16)] = zv
        return 0

    lax.fori_loop(0, rows * per_row, body, 0)


def _make_agg(d):
    mesh = plsc.VectorSubcoreMesh(**_MESH)

    @functools.partial(
        pl.kernel,
        out_type=jax.ShapeDtypeStruct((N2R, d), jnp.float32),
        mesh=mesh,
        scratch_types=[
            pltpu.VMEM((SLAB, CH), jnp.int32),
            pltpu.VMEM((SLAB, CH), jnp.int32),
            pltpu.VMEM((CH, d), jnp.float32),
            pltpu.VMEM((CH, d), jnp.float32),
            pltpu.VMEM((CH, d), jnp.float32),
            pltpu.VMEM_SHARED((N2R, d), jnp.float32),
            pltpu.SemaphoreType.DMA,
            pltpu.SemaphoreType.DMA,
            pltpu.SemaphoreType.DMA,
        ],
    )
    def agg(hs_hbm, src_hbm, dst_hbm, out_hbm,
            src_v, dst_v, bufa, bufb, bufc, acc, sema, semb, semc):
        sid = lax.axis_index("s")
        wid = sid

        _zero_fill(bufa, CH, d)
        for j in range(RPT // CH):
            pltpu.sync_copy(bufa, acc.at[pl.ds(sid * RPT + j * CH, CH)])
        plsc.subcore_barrier()

        bufs = (bufa, bufb, bufc)
        sems = (sema, semb, semc)

        def step(i, b, prefetch):
            pltpu.make_async_copy(hs_hbm.at[src_v.at[i]], bufs[b],
                                  sems[b]).wait()
            pltpu.sync_copy(bufs[b], acc.at[dst_v.at[i]], add=True)
            if prefetch:
                pltpu.async_copy(hs_hbm.at[src_v.at[i + 3]], bufs[b], sems[b])

        for slab in range(NSLAB):
            pltpu.sync_copy(src_hbm.at[wid, slab], src_v)
            pltpu.sync_copy(dst_hbm.at[wid, slab], dst_v)

            for b in range(3):
                pltpu.async_copy(hs_hbm.at[src_v.at[b]], bufs[b], sems[b])
            step(0, 0, True)
            step(1, 1, True)

            def triple(p, _):
                g = 2 + 3 * p
                for b in range(3):
                    step(g + b, (2 + b) % 3, True)
                return 0

            lax.fori_loop(0, (SLAB - 5) // 3, triple, 0)
            for i in range(SLAB - 3, SLAB):
                step(i, i % 3, False)

        plsc.subcore_barrier()
        pltpu.sync_copy(acc.at[pl.ds(sid * RPT, RPT)],
                        out_hbm.at[pl.ds(sid * RPT, RPT)])

    return agg


_agg_h = _make_agg(D_H)


def _make_deg():
    mesh = plsc.VectorSubcoreMesh(**_MESH)

    @functools.partial(
        pl.kernel,
        out_type=jax.ShapeDtypeStruct((N2R,), jnp.float32),
        mesh=mesh,
        scratch_types=[
            pltpu.VMEM((NCHUNK, CH), jnp.int32),
            pltpu.VMEM((CH,), jnp.float32),
            pltpu.VMEM((DPT,), jnp.float32),
            pltpu.VMEM_SHARED((N2R,), jnp.float32),
        ],
    )
    def deg_k(dst_hbm, out_hbm, dst_v, ones_v, zbuf, acc):
        sid = lax.axis_index("s")
        wid = sid

        ov = jnp.ones((16,), jnp.float32)
        for k in range(CH // 16):
            ones_v[pl.ds(k * 16, 16)] = ov
        zv = jnp.zeros((16,), jnp.float32)

        def zb(i, _):
            zbuf[pl.ds(i * 16, 16)] = zv
            return 0

        lax.fori_loop(0, DPT // 16, zb, 0)
        pltpu.sync_copy(zbuf, acc.at[pl.ds(sid * DPT, DPT)])
        pltpu.sync_copy(dst_hbm.at[wid], dst_v)
        plsc.subcore_barrier()

        def body(i, _):
            pltpu.sync_copy(ones_v, acc.at[dst_v.at[i]], add=True)
            return 0

        lax.fori_loop(0, NCHUNK, body, 0)

        plsc.subcore_barrier()
        pltpu.sync_copy(acc.at[pl.ds(sid * DPT, DPT)],
                        out_hbm.at[pl.ds(sid * DPT, DPT)])

    return deg_k


_deg_k = _make_deg()

R_BLK = 10000
GRID = N // R_BLK


def _tc1_body(x_ref, g_ref, b_ref, w_ref, d0_ref, hs_ref, dinv_ref):
    x = x_ref[...]
    mu = jnp.mean(x, axis=1, keepdims=True)
    xc = x - mu
    var = jnp.mean(xc * xc, axis=1, keepdims=True)
    h = xc * lax.rsqrt(var + EPS) * g_ref[...] + b_ref[...]
    dinv = lax.rsqrt(d0_ref[...] + 1.0)
    hs = jnp.dot(h, w_ref[...], preferred_element_type=jnp.float32) * dinv
    hs_ref[...] = hs
    dinv_ref[...] = dinv


def _tc_mid_body(s0_ref, hs_ref, dinv_ref, b_ref, w_ref, out_ref):
    dinv = dinv_ref[...]
    h = (s0_ref[...] + hs_ref[...]) * dinv + b_ref[...]
    h = jnp.maximum(h, 0.0)
    out_ref[...] = jnp.dot(h, w_ref[...],
                           preferred_element_type=jnp.float32) * dinv


def _tc_g3_body(s0_ref, hs_ref, dinv_ref, b_ref, g3_ref):
    dinv = dinv_ref[...]
    h = (s0_ref[...] + hs_ref[...]) * dinv + b_ref[...]
    g3_ref[...] = jnp.maximum(h, 0.0) * dinv


def _tc_fin_body(t_ref, g3_ref, dinv_ref, b_ref, w_ref, z_ref):
    s = t_ref[...] + g3_ref[...]
    z_ref[...] = (jnp.dot(s, w_ref[...], preferred_element_type=jnp.float32)
                  * dinv_ref[...] + b_ref[...])


def _row_spec(cols):
    return pl.BlockSpec((R_BLK, cols), lambda i: (i, 0))


def _full_spec(rows, cols):
    return pl.BlockSpec((rows, cols), lambda i: (0, 0))


def _tc1(x, g, b, w, d0):
    return pl.pallas_call(
        _tc1_body,
        grid=(GRID,),
        in_specs=[_row_spec(D_IN), _full_spec(1, D_IN), _full_spec(1, D_IN),
                  _full_spec(D_IN, D_H), _row_spec(1)],
        out_specs=[_row_spec(D_H), _row_spec(1)],
        out_shape=[jax.ShapeDtypeStruct((N, D_H), jnp.float32),
                   jax.ShapeDtypeStruct((N, 1), jnp.float32)],
    )(x, g, b, w, d0)


def _tc_mid(s0, hs, dinv, b, w):
    dout = w.shape[1]
    return pl.pallas_call(
        _tc_mid_body,
        grid=(GRID,),
        in_specs=[_row_spec(D_H), _row_spec(D_H), _row_spec(1),
                  _full_spec(1, D_H), _full_spec(D_H, dout)],
        out_specs=_row_spec(dout),
        out_shape=jax.ShapeDtypeStruct((N, dout), jnp.float32),
    )(s0, hs, dinv, b, w)


def _tc_g3(s0, hs, dinv, b):
    return pl.pallas_call(
        _tc_g3_body,
        grid=(GRID,),
        in_specs=[_row_spec(D_H), _row_spec(D_H), _row_spec(1),
                  _full_spec(1, D_H)],
        out_specs=_row_spec(D_H),
        out_shape=jax.ShapeDtypeStruct((N, D_H), jnp.float32),
    )(s0, hs, dinv, b)


def _tc_fin(t, g3, dinv, b, w):
    return pl.pallas_call(
        _tc_fin_body,
        grid=(GRID,),
        in_specs=[_row_spec(D_H), _row_spec(D_H), _row_spec(1),
                  _full_spec(1, D_Z), _full_spec(D_H, D_Z)],
        out_specs=_row_spec(D_Z),
        out_shape=jax.ShapeDtypeStruct((N, D_Z), jnp.float32),
    )(t, g3, dinv, b, w)


def kernel(x, edge_index, ln_g, ln_b, W1, b1, W2, b2, W3, b3):
    src4 = edge_index[0].reshape(NW, NSLAB, SLAB, CH)
    dst4 = edge_index[1].reshape(NW, NSLAB, SLAB, CH)
    dst3 = edge_index[1].reshape(NW, NCHUNK, CH)

    deg = _deg_k(dst3)
    d0 = deg[:N, None]

    hs1, dinv = _tc1(x, ln_g.reshape(1, -1), ln_b.reshape(1, -1), W1, d0)
    s1 = _agg_h(hs1, src4, dst4)
    hs2 = _tc_mid(s1[:N], hs1, dinv, b1.reshape(1, -1), W2)
    s2 = _agg_h(hs2, src4, dst4)
    g3 = _tc_g3(s2[:N], hs2, dinv, b2.reshape(1, -1))
    t3 = _agg_h(g3, src4, dst4)
    return _tc_fin(t3[:N], g3, dinv, b3.reshape(1, -1), W3)
